# Initial kernel scaffold; baseline (speedup 1.0000x reference)
import jax
import jax.numpy as jnp
from jax.experimental import pallas as pl


def kernel(x, t0, t1, t2, t3, W, b):
    raise NotImplementedError("write your pallas kernel here")


# speedup vs baseline: 2.4271x; 2.4271x over previous
import functools

import jax
import jax.numpy as jnp
from jax import lax
from jax.experimental import pallas as pl
from jax.experimental.pallas import tpu as pltpu
from jax.experimental.pallas import tpu_sc as plsc

_B = 16 * 2048
_EMB = (128, 256, 128, 512)
_D = 1024
_NW = 32
_BPW = _B // _NW
_CH = 128


def _sc_gather_body(x0, x1, x2, x3, t0, t1, t2, t3,
                    e0, e1, e2, e3,
                    idx_v, buf_a, buf_b, buf_c, sem):
    wid = lax.axis_index("s") * 2 + lax.axis_index("c")
    base = wid * _BPW
    nch = _BPW // _CH

    def run_table(xi, ti, ei, buf):
        def body(j, carry):
            off = base + j * _CH
            pltpu.sync_copy(xi.at[pl.ds(off, _CH)], idx_v)
            pltpu.async_copy(ti.at[idx_v], buf, sem).wait()
            pltpu.sync_copy(buf, ei.at[pl.ds(off, _CH)])
            return carry
        lax.fori_loop(0, nch, body, 0)

    run_table(x0, t0, e0, buf_a)
    run_table(x2, t2, e2, buf_a)
    run_table(x1, t1, e1, buf_b)
    run_table(x3, t3, e3, buf_c)


def _sc_gather(x0, x1, x2, x3, t0, t1, t2, t3):
    mesh = plsc.VectorSubcoreMesh(core_axis_name="c", subcore_axis_name="s")
    k = pl.kernel(
        _sc_gather_body,
        out_type=[jax.ShapeDtypeStruct((_B, e), jnp.float32) for e in _EMB],
        mesh=mesh,
        scratch_types=[
            pltpu.VMEM((_CH,), jnp.int32),
            pltpu.VMEM((_CH, 128), jnp.float32),
            pltpu.VMEM((_CH, 256), jnp.float32),
            pltpu.VMEM((_CH, 512), jnp.float32),
            pltpu.SemaphoreType.DMA,
        ],
    )
    return k(x0, x1, x2, x3, t0, t1, t2, t3)


def _mm_body(e0, e1, e2, e3, w, bias, out):
    acc = jnp.dot(e0[...], w[0:128, :], preferred_element_type=jnp.float32)
    acc = acc + jnp.dot(e1[...], w[128:384, :], preferred_element_type=jnp.float32)
    acc = acc + jnp.dot(e2[...], w[384:512, :], preferred_element_type=jnp.float32)
    acc = acc + jnp.dot(e3[...], w[512:1024, :], preferred_element_type=jnp.float32)
    out[...] = acc + bias[...]


def _tc_matmul(e0, e1, e2, e3, W, b):
    bm = 2048
    grid = (_B // bm,)
    return pl.pallas_call(
        _mm_body,
        grid=grid,
        in_specs=[
            pl.BlockSpec((bm, _EMB[0]), lambda i: (i, 0)),
            pl.BlockSpec((bm, _EMB[1]), lambda i: (i, 0)),
            pl.BlockSpec((bm, _EMB[2]), lambda i: (i, 0)),
            pl.BlockSpec((bm, _EMB[3]), lambda i: (i, 0)),
            pl.BlockSpec((sum(_EMB), _D), lambda i: (0, 0)),
            pl.BlockSpec((1, _D), lambda i: (0, 0)),
        ],
        out_specs=pl.BlockSpec((bm, _D), lambda i: (i, 0)),
        out_shape=jax.ShapeDtypeStruct((_B, _D), jnp.float32),
        compiler_params=pltpu.CompilerParams(
            dimension_semantics=("arbitrary",),
        ),
    )(e0, e1, e2, e3, W, b.reshape(1, _D))


def kernel(x, t0, t1, t2, t3, W, b):
    lead = x.shape[:-1]
    xr = x.reshape(-1, 4).astype(jnp.int32)
    x0, x1, x2, x3 = xr[:, 0], xr[:, 1], xr[:, 2], xr[:, 3]
    e0, e1, e2, e3 = _sc_gather(x0, x1, x2, x3, t0, t1, t2, t3)
    out = _tc_matmul(e0, e1, e2, e3, W, b)
    return out.reshape(*lead, _D)
